# baseline (speedup 1.0000x reference)
import jax
import jax.numpy as jnp
from jax.experimental import pallas as pl


def _body(vid_ref, sid_ref, ve_ref, se_ref, tok_ref, out_ref):
    vid = vid_ref[...]
    sid = sid_ref[...]
    ve = ve_ref[...]
    se = se_ref[...]
    vmask = (vid == 1).astype(jnp.float32)
    smask = (sid == 1).astype(jnp.float32)
    geom = (ve[0][None, :] + vmask * (ve[1] - ve[0])[None, :]
            + se[0][None, :] + smask * (se[1] - se[0])[None, :])
    out_ref[...] = tok_ref[...] + geom[:, None, :]


def kernel(tokens, view_ids, side_ids, view_embed, side_embed):
    B, L, D = tokens.shape
    BB = 16
    vid2d = view_ids.astype(jnp.int32).reshape(B, 1)
    sid2d = side_ids.astype(jnp.int32).reshape(B, 1)
    grid = (B // BB,)
    return pl.pallas_call(
        _body,
        grid=grid,
        in_specs=[
            pl.BlockSpec((BB, 1), lambda i: (i, 0)),
            pl.BlockSpec((BB, 1), lambda i: (i, 0)),
            pl.BlockSpec((2, D), lambda i: (0, 0)),
            pl.BlockSpec((2, D), lambda i: (0, 0)),
            pl.BlockSpec((BB, L, D), lambda i: (i, 0, 0)),
        ],
        out_specs=pl.BlockSpec((BB, L, D), lambda i: (i, 0, 0)),
        out_shape=jax.ShapeDtypeStruct((B, L, D), tokens.dtype),
    )(vid2d, sid2d, view_embed, side_embed, tokens)


# speedup vs baseline: 1.2181x; 1.2181x over previous
import jax
import jax.numpy as jnp
from jax.experimental import pallas as pl


def _body(vid_ref, sid_ref, ve_ref, se_ref, tok_ref, out_ref):
    vid = vid_ref[...]
    sid = sid_ref[...]
    ve = ve_ref[...]
    se = se_ref[...]
    vmask = (vid == 1).astype(jnp.float32)
    smask = (sid == 1).astype(jnp.float32)
    geom = (ve[0][None, :] + vmask * (ve[1] - ve[0])[None, :]
            + se[0][None, :] + smask * (se[1] - se[0])[None, :])
    out_ref[...] = tok_ref[...] + geom[:, None, :]


def kernel(tokens, view_ids, side_ids, view_embed, side_embed):
    B, L, D = tokens.shape
    BB = 64
    vid2d = view_ids.astype(jnp.int32).reshape(B, 1)
    sid2d = side_ids.astype(jnp.int32).reshape(B, 1)
    grid = (B // BB,)
    return pl.pallas_call(
        _body,
        grid=grid,
        in_specs=[
            pl.BlockSpec((BB, 1), lambda i: (i, 0)),
            pl.BlockSpec((BB, 1), lambda i: (i, 0)),
            pl.BlockSpec((2, D), lambda i: (0, 0)),
            pl.BlockSpec((2, D), lambda i: (0, 0)),
            pl.BlockSpec((BB, L, D), lambda i: (i, 0, 0)),
        ],
        out_specs=pl.BlockSpec((BB, L, D), lambda i: (i, 0, 0)),
        out_shape=jax.ShapeDtypeStruct((B, L, D), tokens.dtype),
    )(vid2d, sid2d, view_embed, side_embed, tokens)


# speedup vs baseline: 1.2236x; 1.0045x over previous
import jax
import jax.numpy as jnp
from jax.experimental import pallas as pl


def _body(vid_ref, sid_ref, ve_ref, se_ref, tok_ref, out_ref):
    vid = vid_ref[...]
    sid = sid_ref[...]
    ve = ve_ref[...]
    se = se_ref[...]
    vmask = (vid == 1).astype(jnp.float32)
    smask = (sid == 1).astype(jnp.float32)
    geom = (ve[0][None, :] + vmask * (ve[1] - ve[0])[None, :]
            + se[0][None, :] + smask * (se[1] - se[0])[None, :])
    out_ref[...] = tok_ref[...] + geom[:, None, :]


def kernel(tokens, view_ids, side_ids, view_embed, side_embed):
    B, L, D = tokens.shape
    BB = 128
    vid2d = view_ids.astype(jnp.int32).reshape(B, 1)
    sid2d = side_ids.astype(jnp.int32).reshape(B, 1)
    grid = (B // BB,)
    return pl.pallas_call(
        _body,
        grid=grid,
        in_specs=[
            pl.BlockSpec((BB, 1), lambda i: (i, 0)),
            pl.BlockSpec((BB, 1), lambda i: (i, 0)),
            pl.BlockSpec((2, D), lambda i: (0, 0)),
            pl.BlockSpec((2, D), lambda i: (0, 0)),
            pl.BlockSpec((BB, L, D), lambda i: (i, 0, 0)),
        ],
        out_specs=pl.BlockSpec((BB, L, D), lambda i: (i, 0, 0)),
        out_shape=jax.ShapeDtypeStruct((B, L, D), tokens.dtype),
    )(vid2d, sid2d, view_embed, side_embed, tokens)
